# baseline (speedup 1.0000x reference)
import jax
import jax.numpy as jnp
from jax import lax
from jax.experimental import pallas as pl
from jax.experimental.pallas import tpu as pltpu

_B, _N1, _N2 = 16, 1024, 1024
_RB = 512
_NBLK = _N1 // _RB


def _cw_body(tgt_ref, src_ref, pred_ref, gt_ref, out_ref, acc_ref):
    b = pl.program_id(0)
    r = pl.program_id(1)
    nt = tgt_ref[b]
    ns = src_ref[b]
    px = pred_ref[0]
    gx = gt_ref[0]
    neg = jnp.float32(-jnp.inf)

    col1 = lax.broadcasted_iota(jnp.int32, (1, _N2), 1)
    maskrow = jnp.where(col1 < nt, 0.0, neg)

    mg = gx + maskrow
    g1 = jnp.max(mg, axis=1, keepdims=True)
    col = lax.broadcasted_iota(jnp.int32, (_RB, _N2), 1)
    gidx = jnp.min(jnp.where(mg == g1, col, _N2), axis=1, keepdims=True)

    mp = px + maskrow
    h = jnp.maximum(mp[:, :512], mp[:, 512:])
    l = jnp.minimum(mp[:, :512], mp[:, 512:])
    for w in (256, 128):
        h1, h2 = h[:, :w], h[:, w:]
        l = jnp.maximum(jnp.minimum(h1, h2), jnp.maximum(l[:, :w], l[:, w:]))
        h = jnp.maximum(h1, h2)
    m1 = jnp.max(h, axis=1, keepdims=True)
    is_m1 = h == m1
    z = jnp.maximum(jnp.where(is_m1, neg, h), l)
    m2_strict = jnp.max(z, axis=1, keepdims=True)
    dup = jnp.sum(is_m1, axis=1, keepdims=True) > 1
    m2 = jnp.where(dup, m1, m2_strict)

    grp = lax.shift_right_logical(gidx, 7)
    strip = px[:, :128]
    for g in range(1, 8):
        strip = jnp.where(grp == g, px[:, g * 128 : (g + 1) * 128], strip)
    lane = lax.bitwise_and(gidx, 127)
    col128 = lax.broadcasted_iota(jnp.int32, (_RB, 128), 1)
    pag = jnp.max(jnp.where(col128 == lane, strip, neg), axis=1, keepdims=True)

    sel = jnp.where(pag == m1, m2, m1)
    contrib = jnp.log(sel) - jnp.log(pag)
    row = r * _RB + lax.broadcasted_iota(jnp.int32, (_RB, 1), 0)
    partial = jnp.sum(jnp.where(row < ns, contrib, 0.0))

    first = jnp.logical_and(b == 0, r == 0)
    acc_ref[0] = jnp.where(first, 0.0, acc_ref[0]) + partial

    @pl.when(jnp.logical_and(b == _B - 1, r == _NBLK - 1))
    def _():
        n_sum = lax.fori_loop(
            0, _B, lambda i, s: s + src_ref[i].astype(jnp.float32), jnp.float32(0.0)
        )
        out_ref[0, 0] = acc_ref[0] / n_sum


def kernel(pred_dsmat, gt_perm, src_ns, tgt_ns):
    pred_dsmat = pred_dsmat.astype(jnp.float32)
    gt_perm = gt_perm.astype(jnp.float32)

    out = pl.pallas_call(
        _cw_body,
        grid=(_B, _NBLK),
        in_specs=[
            pl.BlockSpec(memory_space=pltpu.SMEM),
            pl.BlockSpec(memory_space=pltpu.SMEM),
            pl.BlockSpec((1, _RB, _N2), lambda b, r: (b, r, 0)),
            pl.BlockSpec((1, _RB, _N2), lambda b, r: (b, r, 0)),
        ],
        out_specs=pl.BlockSpec(memory_space=pltpu.SMEM),
        out_shape=jax.ShapeDtypeStruct((1, 1), jnp.float32),
        scratch_shapes=[pltpu.SMEM((1,), jnp.float32)],
    )(tgt_ns, src_ns, pred_dsmat, gt_perm)
    return out[0, 0]


# speedup vs baseline: 1.1491x; 1.1491x over previous
import jax
import jax.numpy as jnp
from jax import lax
from jax.experimental import pallas as pl
from jax.experimental.pallas import tpu as pltpu

_B, _N1, _N2 = 16, 1024, 1024
_RB = 1024
_NBLK = _N1 // _RB


def _cw_body(tgt_ref, src_ref, pred_ref, gt_ref, out_ref, acc_ref):
    b = pl.program_id(0)
    r = pl.program_id(1)
    nt = tgt_ref[b]
    ns = src_ref[b]
    px = pred_ref[0]
    gx = gt_ref[0]
    neg = jnp.float32(-jnp.inf)

    col1 = lax.broadcasted_iota(jnp.int32, (1, _N2), 1)
    maskrow = jnp.where(col1 < nt, 0.0, neg)

    mg = gx + maskrow
    g1 = jnp.max(mg, axis=1, keepdims=True)
    col = lax.broadcasted_iota(jnp.int32, (_RB, _N2), 1)
    gidx = jnp.min(jnp.where(mg == g1, col, _N2), axis=1, keepdims=True)

    mp = px + maskrow
    h = jnp.maximum(mp[:, :512], mp[:, 512:])
    l = jnp.minimum(mp[:, :512], mp[:, 512:])
    for w in (256, 128):
        h1, h2 = h[:, :w], h[:, w:]
        l = jnp.maximum(jnp.minimum(h1, h2), jnp.maximum(l[:, :w], l[:, w:]))
        h = jnp.maximum(h1, h2)
    m1 = jnp.max(h, axis=1, keepdims=True)
    is_m1 = h == m1
    z = jnp.maximum(jnp.where(is_m1, neg, h), l)
    m2_strict = jnp.max(z, axis=1, keepdims=True)
    dup = jnp.sum(is_m1, axis=1, keepdims=True) > 1
    m2 = jnp.where(dup, m1, m2_strict)

    grp = lax.shift_right_logical(gidx, 7)
    strip = px[:, :128]
    for g in range(1, 8):
        strip = jnp.where(grp == g, px[:, g * 128 : (g + 1) * 128], strip)
    lane = lax.bitwise_and(gidx, 127)
    col128 = lax.broadcasted_iota(jnp.int32, (_RB, 128), 1)
    pag = jnp.max(jnp.where(col128 == lane, strip, neg), axis=1, keepdims=True)

    sel = jnp.where(pag == m1, m2, m1)
    contrib = jnp.log(sel) - jnp.log(pag)
    row = r * _RB + lax.broadcasted_iota(jnp.int32, (_RB, 1), 0)
    partial = jnp.sum(jnp.where(row < ns, contrib, 0.0))

    first = jnp.logical_and(b == 0, r == 0)
    acc_ref[0] = jnp.where(first, 0.0, acc_ref[0]) + partial

    @pl.when(jnp.logical_and(b == _B - 1, r == _NBLK - 1))
    def _():
        n_sum = lax.fori_loop(
            0, _B, lambda i, s: s + src_ref[i].astype(jnp.float32), jnp.float32(0.0)
        )
        out_ref[0, 0] = acc_ref[0] / n_sum


def kernel(pred_dsmat, gt_perm, src_ns, tgt_ns):
    pred_dsmat = pred_dsmat.astype(jnp.float32)
    gt_perm = gt_perm.astype(jnp.float32)

    out = pl.pallas_call(
        _cw_body,
        grid=(_B, _NBLK),
        in_specs=[
            pl.BlockSpec(memory_space=pltpu.SMEM),
            pl.BlockSpec(memory_space=pltpu.SMEM),
            pl.BlockSpec((1, _RB, _N2), lambda b, r: (b, r, 0)),
            pl.BlockSpec((1, _RB, _N2), lambda b, r: (b, r, 0)),
        ],
        out_specs=pl.BlockSpec(memory_space=pltpu.SMEM),
        out_shape=jax.ShapeDtypeStruct((1, 1), jnp.float32),
        scratch_shapes=[pltpu.SMEM((1,), jnp.float32)],
    )(tgt_ns, src_ns, pred_dsmat, gt_perm)
    return out[0, 0]


# speedup vs baseline: 1.2159x; 1.0582x over previous
import jax
import jax.numpy as jnp
from jax import lax
from jax.experimental import pallas as pl
from jax.experimental.pallas import tpu as pltpu

_B, _N1, _N2 = 16, 1024, 1024
_BPS = 2
_NSTEP = _B // _BPS


def _one_batch(nt, ns, px, gx):
    neg = jnp.float32(-jnp.inf)
    col1 = lax.broadcasted_iota(jnp.int32, (1, _N2), 1)
    maskrow = jnp.where(col1 < nt, 0.0, neg)

    mg = gx + maskrow
    g1 = jnp.max(mg, axis=1, keepdims=True)
    col = lax.broadcasted_iota(jnp.int32, (_N1, _N2), 1)
    gidx = jnp.min(jnp.where(mg == g1, col, _N2), axis=1, keepdims=True)

    mp = px + maskrow
    h = jnp.maximum(mp[:, :512], mp[:, 512:])
    l = jnp.minimum(mp[:, :512], mp[:, 512:])
    for w in (256, 128):
        h1, h2 = h[:, :w], h[:, w:]
        l = jnp.maximum(jnp.minimum(h1, h2), jnp.maximum(l[:, :w], l[:, w:]))
        h = jnp.maximum(h1, h2)
    m1 = jnp.max(h, axis=1, keepdims=True)
    is_m1 = h == m1
    z = jnp.maximum(jnp.where(is_m1, neg, h), l)
    m2_strict = jnp.max(z, axis=1, keepdims=True)
    dup = jnp.sum(is_m1, axis=1, keepdims=True) > 1
    m2 = jnp.where(dup, m1, m2_strict)

    grp = lax.shift_right_logical(gidx, 7)
    strip = px[:, :128]
    for g in range(1, 8):
        strip = jnp.where(grp == g, px[:, g * 128 : (g + 1) * 128], strip)
    lane = lax.bitwise_and(gidx, 127)
    col128 = lax.broadcasted_iota(jnp.int32, (_N1, 128), 1)
    pag = jnp.max(jnp.where(col128 == lane, strip, neg), axis=1, keepdims=True)

    sel = jnp.where(pag == m1, m2, m1)
    contrib = jnp.log(sel) - jnp.log(pag)
    row = lax.broadcasted_iota(jnp.int32, (_N1, 1), 0)
    return jnp.sum(jnp.where(row < ns, contrib, 0.0))


def _cw_body(tgt_ref, src_ref, pred_ref, gt_ref, out_ref, acc_ref):
    s = pl.program_id(0)
    partial = jnp.float32(0.0)
    for k in range(_BPS):
        b = s * _BPS + k
        partial += _one_batch(tgt_ref[b], src_ref[b], pred_ref[k], gt_ref[k])

    acc_ref[0] = jnp.where(s == 0, 0.0, acc_ref[0]) + partial

    @pl.when(s == _NSTEP - 1)
    def _():
        n_sum = lax.fori_loop(
            0, _B, lambda i, v: v + src_ref[i].astype(jnp.float32), jnp.float32(0.0)
        )
        out_ref[0, 0] = acc_ref[0] / n_sum


def kernel(pred_dsmat, gt_perm, src_ns, tgt_ns):
    pred_dsmat = pred_dsmat.astype(jnp.float32)
    gt_perm = gt_perm.astype(jnp.float32)

    out = pl.pallas_call(
        _cw_body,
        grid=(_NSTEP,),
        in_specs=[
            pl.BlockSpec(memory_space=pltpu.SMEM),
            pl.BlockSpec(memory_space=pltpu.SMEM),
            pl.BlockSpec((_BPS, _N1, _N2), lambda s: (s, 0, 0)),
            pl.BlockSpec((_BPS, _N1, _N2), lambda s: (s, 0, 0)),
        ],
        out_specs=pl.BlockSpec(memory_space=pltpu.SMEM),
        out_shape=jax.ShapeDtypeStruct((1, 1), jnp.float32),
        scratch_shapes=[pltpu.SMEM((1,), jnp.float32)],
    )(tgt_ns, src_ns, pred_dsmat, gt_perm)
    return out[0, 0]


# speedup vs baseline: 1.2603x; 1.0365x over previous
import jax
import jax.numpy as jnp
from jax import lax
from jax.experimental import pallas as pl
from jax.experimental.pallas import tpu as pltpu

_B, _N1, _N2 = 16, 1024, 1024
_BPS = 2
_NSTEP = _B // _BPS


def _one_batch(nt, ns, px, gx):
    neg = jnp.float32(-jnp.inf)
    col1 = lax.broadcasted_iota(jnp.int32, (1, _N2), 1)
    maskrow = jnp.where(col1 < nt, 0.0, neg)

    mg = gx + maskrow
    g1 = jnp.max(mg, axis=1, keepdims=True)
    tfirst = jnp.full((_N1, 128), 8, jnp.int32)
    for t in range(7, -1, -1):
        tfirst = jnp.where(mg[:, t * 128 : (t + 1) * 128] == g1, t, tfirst)
    lane128 = lax.broadcasted_iota(jnp.int32, (_N1, 128), 1)
    cand = tfirst * 128 + lane128
    gidx = jnp.min(cand, axis=1, keepdims=True)

    mp = px + maskrow
    h = jnp.maximum(mp[:, :512], mp[:, 512:])
    l = jnp.minimum(mp[:, :512], mp[:, 512:])
    for w in (256, 128):
        h1, h2 = h[:, :w], h[:, w:]
        l = jnp.maximum(jnp.minimum(h1, h2), jnp.maximum(l[:, :w], l[:, w:]))
        h = jnp.maximum(h1, h2)
    m1 = jnp.max(h, axis=1, keepdims=True)
    is_m1 = h == m1
    z = jnp.maximum(jnp.where(is_m1, neg, h), l)
    m2_strict = jnp.max(z, axis=1, keepdims=True)
    dup = jnp.sum(is_m1, axis=1, keepdims=True) > 1
    m2 = jnp.where(dup, m1, m2_strict)

    grp = lax.shift_right_logical(gidx, 7)
    strip = px[:, :128]
    for g in range(1, 8):
        strip = jnp.where(grp == g, px[:, g * 128 : (g + 1) * 128], strip)
    lane = lax.bitwise_and(gidx, 127)
    col128 = lax.broadcasted_iota(jnp.int32, (_N1, 128), 1)
    pag = jnp.max(jnp.where(col128 == lane, strip, neg), axis=1, keepdims=True)

    sel = jnp.where(pag == m1, m2, m1)
    contrib = jnp.log(sel / pag)
    row = lax.broadcasted_iota(jnp.int32, (_N1, 1), 0)
    return jnp.sum(jnp.where(row < ns, contrib, 0.0))


def _cw_body(tgt_ref, src_ref, pred_ref, gt_ref, out_ref, acc_ref):
    s = pl.program_id(0)
    partial = jnp.float32(0.0)
    for k in range(_BPS):
        b = s * _BPS + k
        partial += _one_batch(tgt_ref[b], src_ref[b], pred_ref[k], gt_ref[k])

    acc_ref[0] = jnp.where(s == 0, 0.0, acc_ref[0]) + partial

    @pl.when(s == _NSTEP - 1)
    def _():
        n_sum = lax.fori_loop(
            0, _B, lambda i, v: v + src_ref[i].astype(jnp.float32), jnp.float32(0.0)
        )
        out_ref[0, 0] = acc_ref[0] / n_sum


def kernel(pred_dsmat, gt_perm, src_ns, tgt_ns):
    pred_dsmat = pred_dsmat.astype(jnp.float32)
    gt_perm = gt_perm.astype(jnp.float32)

    out = pl.pallas_call(
        _cw_body,
        grid=(_NSTEP,),
        in_specs=[
            pl.BlockSpec(memory_space=pltpu.SMEM),
            pl.BlockSpec(memory_space=pltpu.SMEM),
            pl.BlockSpec((_BPS, _N1, _N2), lambda s: (s, 0, 0)),
            pl.BlockSpec((_BPS, _N1, _N2), lambda s: (s, 0, 0)),
        ],
        out_specs=pl.BlockSpec(memory_space=pltpu.SMEM),
        out_shape=jax.ShapeDtypeStruct((1, 1), jnp.float32),
        scratch_shapes=[pltpu.SMEM((1,), jnp.float32)],
    )(tgt_ns, src_ns, pred_dsmat, gt_perm)
    return out[0, 0]


# speedup vs baseline: 1.2968x; 1.0290x over previous
import jax
import jax.numpy as jnp
from jax import lax
from jax.experimental import pallas as pl
from jax.experimental.pallas import tpu as pltpu

_B, _N1, _N2 = 16, 1024, 1024
_BPS = 2
_NSTEP = _B // _BPS


def _one_batch(nt, ns, px, gx):
    neg = jnp.float32(-jnp.inf)
    col1 = lax.broadcasted_iota(jnp.int32, (1, _N2), 1)
    maskrow = jnp.where(col1 < nt, 0.0, neg)

    mg = gx + maskrow
    g1 = jnp.max(mg, axis=1, keepdims=True)
    tfirst = jnp.full((_N1, 128), 8, jnp.int32)
    for t in range(7, -1, -1):
        tfirst = jnp.where(mg[:, t * 128 : (t + 1) * 128] == g1, t, tfirst)
    lane128 = lax.broadcasted_iota(jnp.int32, (_N1, 128), 1)
    cand = tfirst * 128 + lane128
    gidx = jnp.min(cand, axis=1, keepdims=True)

    mp = px + maskrow
    h = jnp.maximum(mp[:, :512], mp[:, 512:])
    l = jnp.minimum(mp[:, :512], mp[:, 512:])
    for w in (256, 128):
        h1, h2 = h[:, :w], h[:, w:]
        l = jnp.maximum(jnp.minimum(h1, h2), jnp.maximum(l[:, :w], l[:, w:]))
        h = jnp.maximum(h1, h2)
    m1 = jnp.max(h, axis=1, keepdims=True)
    is_m1 = h == m1
    z = jnp.maximum(jnp.where(is_m1, neg, h), l)
    m2_strict = jnp.max(z, axis=1, keepdims=True)
    dup = jnp.sum(jnp.where(is_m1, 1.0, 0.0), axis=1, keepdims=True) > 1.5
    m2 = jnp.where(dup, m1, m2_strict)

    grp = lax.shift_right_logical(gidx, 7)
    strip = px[:, :128]
    for g in range(1, 8):
        strip = jnp.where(grp == g, px[:, g * 128 : (g + 1) * 128], strip)
    lane = lax.bitwise_and(gidx, 127)
    col128 = lax.broadcasted_iota(jnp.int32, (_N1, 128), 1)
    pag = jnp.max(jnp.where(col128 == lane, strip, neg), axis=1, keepdims=True)

    sel = jnp.where(pag == m1, m2, m1)
    contrib = jnp.log(sel / pag)
    row = lax.broadcasted_iota(jnp.int32, (_N1, 1), 0)
    return jnp.sum(jnp.where(row < ns, contrib, 0.0))


def _cw_body(tgt_ref, src_ref, pred_ref, gt_ref, out_ref, acc_ref):
    s = pl.program_id(0)
    partial = jnp.float32(0.0)
    for k in range(_BPS):
        b = s * _BPS + k
        partial += _one_batch(tgt_ref[b], src_ref[b], pred_ref[k], gt_ref[k])

    acc_ref[0] = jnp.where(s == 0, 0.0, acc_ref[0]) + partial

    @pl.when(s == _NSTEP - 1)
    def _():
        n_sum = lax.fori_loop(
            0, _B, lambda i, v: v + src_ref[i].astype(jnp.float32), jnp.float32(0.0)
        )
        out_ref[0, 0] = acc_ref[0] / n_sum


def kernel(pred_dsmat, gt_perm, src_ns, tgt_ns):
    pred_dsmat = pred_dsmat.astype(jnp.float32)
    gt_perm = gt_perm.astype(jnp.float32)

    out = pl.pallas_call(
        _cw_body,
        grid=(_NSTEP,),
        in_specs=[
            pl.BlockSpec(memory_space=pltpu.SMEM),
            pl.BlockSpec(memory_space=pltpu.SMEM),
            pl.BlockSpec((_BPS, _N1, _N2), lambda s: (s, 0, 0)),
            pl.BlockSpec((_BPS, _N1, _N2), lambda s: (s, 0, 0)),
        ],
        out_specs=pl.BlockSpec(memory_space=pltpu.SMEM),
        out_shape=jax.ShapeDtypeStruct((1, 1), jnp.float32),
        scratch_shapes=[pltpu.SMEM((1,), jnp.float32)],
    )(tgt_ns, src_ns, pred_dsmat, gt_perm)
    return out[0, 0]
